# baseline (speedup 1.0000x reference)
import functools

import jax
import jax.numpy as jnp
import numpy as np
from jax import lax
from jax.experimental import pallas as pl
from jax.experimental.pallas import tpu as pltpu
from jax.experimental.pallas import tpu_sc as plsc

N_CP = 20
L = 16
NC, NS = 2, 16
NW = NC * NS

_S_VALS = tuple(float(v) for v in np.linspace(0.0, 1.0, N_CP + 1)[1:].astype(np.float32))


@functools.lru_cache(maxsize=None)
def _make_sc_kernel(B: int, n_items: int):
  b_per_w = B // NW
  n_chunks = b_per_w // L
  mesh = plsc.VectorSubcoreMesh(
      core_axis_name="c", subcore_axis_name="s", num_cores=NC, num_subcores=NS)

  @functools.partial(
      pl.kernel,
      mesh=mesh,
      compiler_params=pltpu.CompilerParams(
          needs_layout_passes=False, use_tc_tiling_on_sc=False),
      out_type=jax.ShapeDtypeStruct((B,), jnp.float32),
      scratch_types=[
          pltpu.VMEM((b_per_w,), jnp.int32),
          pltpu.VMEM((b_per_w,), jnp.float32),
          pltpu.VMEM((b_per_w,), jnp.float32),
          pltpu.VMEM((b_per_w,), jnp.float32),
          pltpu.VMEM((N_CP * b_per_w,), jnp.int32),
          pltpu.VMEM((N_CP * b_per_w,), jnp.float32),
          pltpu.VMEM((b_per_w,), jnp.float32),
          pltpu.SemaphoreType.DMA,
          pltpu.SemaphoreType.DMA,
          pltpu.SemaphoreType.DMA,
      ],
  )
  def trend_kernel(t_hbm, ids_hbm, m_hbm, k_hbm, d_hbm, out_hbm,
                   idx_v, t_v, m_v, k_v, idx2_v, d_v, o_v, sem_m, sem_k, sem_d):
    wid = lax.axis_index("s") * NC + lax.axis_index("c")
    base = wid * b_per_w
    pltpu.sync_copy(ids_hbm.at[pl.ds(base, b_per_w)], idx_v)
    cm = pltpu.async_copy(m_hbm.at[idx_v], m_v, sem_m)
    ck = pltpu.async_copy(k_hbm.at[idx_v], k_v, sem_k)
    pltpu.sync_copy(t_hbm.at[pl.ds(base, b_per_w)], t_v)

    def build_body(c, _):
      off = c * L
      ids_vec = idx_v[pl.ds(off, L)]
      for j in range(N_CP):
        idx2_v[pl.ds(j * b_per_w + off, L)] = ids_vec + j * n_items
      return 0

    lax.fori_loop(0, n_chunks, build_body, 0)
    cd = pltpu.async_copy(d_hbm.at[idx2_v], d_v, sem_d)
    cd.wait()
    cm.wait()
    ck.wait()

    def chunk_body(c, _):
      off = c * L
      tt = t_v[pl.ds(off, L)]
      acc = tt * k_v[pl.ds(off, L)] + m_v[pl.ds(off, L)]
      for j in range(N_CP):
        sj = _S_VALS[j]
        acc = acc + jnp.where(tt > sj, tt - sj, 0.0) * d_v[pl.ds(j * b_per_w + off, L)]
      o_v[pl.ds(off, L)] = acc
      return 0

    lax.fori_loop(0, n_chunks, chunk_body, 0)
    pltpu.sync_copy(o_v, out_hbm.at[pl.ds(base, b_per_w)])

  return trend_kernel


def kernel(t, ids, emb_m, emb_k, emb_delta):
  B = t.shape[0]
  n_items = emb_delta.shape[0]
  dflat = jnp.concatenate([emb_delta[:, j] for j in range(N_CP)])
  out = _make_sc_kernel(B, n_items)(
      t.reshape(B), ids.reshape(B), emb_m.reshape(n_items),
      emb_k.reshape(n_items), dflat)
  return out.reshape(B, 1)


# speedup vs baseline: 6.7536x; 6.7536x over previous
import functools

import jax
import jax.numpy as jnp
import numpy as np
from jax import lax
from jax.experimental import pallas as pl
from jax.experimental.pallas import tpu as pltpu
from jax.experimental.pallas import tpu_sc as plsc

N_CP = 20
L = 16
NC, NS = 2, 16
NW = NC * NS
WC = 65536

_S_VALS = tuple(float(v) for v in np.linspace(0.0, 1.0, N_CP + 1)[1:].astype(np.float32))


@functools.lru_cache(maxsize=None)
def _make_tc_flatten(n_items: int):
  n_chunks = -(-n_items // WC)

  def body(in_ref, out_ref):
    for j in range(N_CP):
      out_ref[pl.ds(j * WC, WC)] = in_ref[j, :]

  return pl.pallas_call(
      body,
      grid=(n_chunks,),
      in_specs=[pl.BlockSpec((N_CP, WC), lambda c: (0, c))],
      out_specs=pl.BlockSpec((N_CP * WC,), lambda c: (c,)),
      out_shape=jax.ShapeDtypeStruct((n_chunks * N_CP * WC,), jnp.float32),
  )


@functools.lru_cache(maxsize=None)
def _make_sc_kernel(B: int, n_items: int):
  b_per_w = B // NW
  n_chunks = b_per_w // L
  mesh = plsc.VectorSubcoreMesh(
      core_axis_name="c", subcore_axis_name="s", num_cores=NC, num_subcores=NS)

  @functools.partial(
      pl.kernel,
      mesh=mesh,
      compiler_params=pltpu.CompilerParams(
          needs_layout_passes=False, use_tc_tiling_on_sc=False),
      out_type=jax.ShapeDtypeStruct((B,), jnp.float32),
      scratch_types=[
          pltpu.VMEM((b_per_w,), jnp.int32),
          pltpu.VMEM((b_per_w,), jnp.float32),
          pltpu.VMEM((b_per_w,), jnp.float32),
          pltpu.VMEM((b_per_w,), jnp.float32),
          pltpu.VMEM((N_CP * b_per_w,), jnp.int32),
          pltpu.VMEM((N_CP * b_per_w,), jnp.float32),
          pltpu.VMEM((b_per_w,), jnp.float32),
          pltpu.SemaphoreType.DMA,
          pltpu.SemaphoreType.DMA,
          pltpu.SemaphoreType.DMA,
      ],
  )
  def trend_kernel(t_hbm, ids_hbm, m_hbm, k_hbm, d_hbm, out_hbm,
                   idx_v, t_v, m_v, k_v, idx2_v, d_v, o_v, sem_m, sem_k, sem_d):
    wid = lax.axis_index("s") * NC + lax.axis_index("c")
    base = wid * b_per_w
    pltpu.sync_copy(ids_hbm.at[pl.ds(base, b_per_w)], idx_v)
    cm = pltpu.async_copy(m_hbm.at[idx_v], m_v, sem_m)
    ck = pltpu.async_copy(k_hbm.at[idx_v], k_v, sem_k)
    pltpu.sync_copy(t_hbm.at[pl.ds(base, b_per_w)], t_v)

    def build_body(c, _):
      off = c * L
      ids_vec = idx_v[pl.ds(off, L)]
      flat0 = (ids_vec >> 16) * (N_CP * WC) + (ids_vec & (WC - 1))
      for j in range(N_CP):
        idx2_v[pl.ds(j * b_per_w + off, L)] = flat0 + j * WC
      return 0

    lax.fori_loop(0, n_chunks, build_body, 0)
    cd = pltpu.async_copy(d_hbm.at[idx2_v], d_v, sem_d)
    cd.wait()
    cm.wait()
    ck.wait()

    def chunk_body(c, _):
      off = c * L
      tt = t_v[pl.ds(off, L)]
      acc = tt * k_v[pl.ds(off, L)] + m_v[pl.ds(off, L)]
      for j in range(N_CP):
        sj = _S_VALS[j]
        acc = acc + jnp.where(tt > sj, tt - sj, 0.0) * d_v[pl.ds(j * b_per_w + off, L)]
      o_v[pl.ds(off, L)] = acc
      return 0

    lax.fori_loop(0, n_chunks, chunk_body, 0)
    pltpu.sync_copy(o_v, out_hbm.at[pl.ds(base, b_per_w)])

  return trend_kernel


def kernel(t, ids, emb_m, emb_k, emb_delta):
  B = t.shape[0]
  n_items = emb_delta.shape[0]
  dflat = _make_tc_flatten(n_items)(emb_delta.T)
  out = _make_sc_kernel(B, n_items)(
      t.reshape(B), ids.reshape(B), emb_m.reshape(n_items),
      emb_k.reshape(n_items), dflat)
  return out.reshape(B, 1)


# speedup vs baseline: 6.7732x; 1.0029x over previous
import functools

import jax
import jax.numpy as jnp
import numpy as np
from jax import lax
from jax.experimental import pallas as pl
from jax.experimental.pallas import tpu as pltpu
from jax.experimental.pallas import tpu_sc as plsc

N_CP = 20
L = 16
NC, NS = 2, 16
NW = NC * NS
WC = 131072
WC_SHIFT = WC.bit_length() - 1

_S_VALS = tuple(float(v) for v in np.linspace(0.0, 1.0, N_CP + 1)[1:].astype(np.float32))


@functools.lru_cache(maxsize=None)
def _make_tc_flatten(n_items: int):
  n_chunks = -(-n_items // WC)

  def body(in_ref, out_ref):
    for j in range(N_CP):
      out_ref[pl.ds(j * WC, WC)] = in_ref[j, :]

  return pl.pallas_call(
      body,
      grid=(n_chunks,),
      in_specs=[pl.BlockSpec((N_CP, WC), lambda c: (0, c))],
      out_specs=pl.BlockSpec((N_CP * WC,), lambda c: (c,)),
      out_shape=jax.ShapeDtypeStruct((n_chunks * N_CP * WC,), jnp.float32),
  )


@functools.lru_cache(maxsize=None)
def _make_sc_kernel(B: int, n_items: int):
  b_per_w = B // NW
  n_chunks = b_per_w // L
  mesh = plsc.VectorSubcoreMesh(
      core_axis_name="c", subcore_axis_name="s", num_cores=NC, num_subcores=NS)

  @functools.partial(
      pl.kernel,
      mesh=mesh,
      compiler_params=pltpu.CompilerParams(
          needs_layout_passes=False, use_tc_tiling_on_sc=False),
      out_type=jax.ShapeDtypeStruct((B,), jnp.float32),
      scratch_types=[
          pltpu.VMEM((b_per_w,), jnp.int32),
          pltpu.VMEM((b_per_w,), jnp.float32),
          pltpu.VMEM((b_per_w,), jnp.float32),
          pltpu.VMEM((b_per_w,), jnp.float32),
          pltpu.VMEM((N_CP * b_per_w,), jnp.int32),
          pltpu.VMEM((N_CP * b_per_w,), jnp.float32),
          pltpu.VMEM((b_per_w,), jnp.float32),
          pltpu.SemaphoreType.DMA,
          pltpu.SemaphoreType.DMA,
          pltpu.SemaphoreType.DMA,
      ],
  )
  def trend_kernel(t_hbm, ids_hbm, m_hbm, k_hbm, d_hbm, out_hbm,
                   idx_v, t_v, m_v, k_v, idx2_v, d_v, o_v, sem_m, sem_k, sem_d):
    wid = lax.axis_index("s") * NC + lax.axis_index("c")
    base = wid * b_per_w
    pltpu.sync_copy(ids_hbm.at[pl.ds(base, b_per_w)], idx_v)
    cm = pltpu.async_copy(m_hbm.at[idx_v], m_v, sem_m)
    ck = pltpu.async_copy(k_hbm.at[idx_v], k_v, sem_k)
    pltpu.sync_copy(t_hbm.at[pl.ds(base, b_per_w)], t_v)

    def build_body(c, _):
      off = c * L
      ids_vec = idx_v[pl.ds(off, L)]
      flat0 = (ids_vec >> WC_SHIFT) * (N_CP * WC) + (ids_vec & (WC - 1))
      for j in range(N_CP):
        idx2_v[pl.ds(j * b_per_w + off, L)] = flat0 + j * WC
      return 0

    lax.fori_loop(0, n_chunks, build_body, 0)
    cd = pltpu.async_copy(d_hbm.at[idx2_v], d_v, sem_d)
    cd.wait()
    cm.wait()
    ck.wait()

    def chunk_body(c, _):
      off = c * L
      tt = t_v[pl.ds(off, L)]
      acc = tt * k_v[pl.ds(off, L)] + m_v[pl.ds(off, L)]
      for j in range(N_CP):
        sj = _S_VALS[j]
        acc = acc + jnp.where(tt > sj, tt - sj, 0.0) * d_v[pl.ds(j * b_per_w + off, L)]
      o_v[pl.ds(off, L)] = acc
      return 0

    lax.fori_loop(0, n_chunks, chunk_body, 0)
    pltpu.sync_copy(o_v, out_hbm.at[pl.ds(base, b_per_w)])

  return trend_kernel


def kernel(t, ids, emb_m, emb_k, emb_delta):
  B = t.shape[0]
  n_items = emb_delta.shape[0]
  dflat = _make_tc_flatten(n_items)(emb_delta.T)
  out = _make_sc_kernel(B, n_items)(
      t.reshape(B), ids.reshape(B), emb_m.reshape(n_items),
      emb_k.reshape(n_items), dflat)
  return out.reshape(B, 1)
